# Initial kernel scaffold; baseline (speedup 1.0000x reference)
import jax
import jax.numpy as jnp
from jax.experimental import pallas as pl


def kernel(image, x):
    raise NotImplementedError("write your pallas kernel here")


# speedup vs baseline: 11.9631x; 11.9631x over previous
import functools

import jax
import jax.numpy as jnp
from jax import lax
from jax.experimental import pallas as pl
from jax.experimental.pallas import tpu as pltpu
from jax.experimental.pallas import tpu_sc as plsc

_L = 16
_CH = 1024
_G = _CH // 128


def _interp_kernel(n, c, nw):
    n_per_w = n // nw
    n_chunks = n_per_w // _CH
    mesh = plsc.VectorSubcoreMesh(core_axis_name="c", subcore_axis_name="s")

    @functools.partial(
        pl.kernel,
        mesh=mesh,
        compiler_params=pltpu.CompilerParams(use_tc_tiling_on_sc=False),
        out_type=jax.ShapeDtypeStruct((n, c), jnp.float32),
        scratch_types=[
            pltpu.VMEM((_CH,), jnp.float32),
            pltpu.VMEM((_CH,), jnp.float32),
            pltpu.VMEM((_G, 128), jnp.int32),
            pltpu.VMEM((_CH,), jnp.float32),
            pltpu.VMEM((_CH, c), jnp.float32),
            pltpu.SemaphoreType.DMA,
        ],
    )
    def body(table_hbm, x0_hbm, x1_hbm, out_hbm, x0_v, x1_v, idx_v, w_v,
             rows_v, sem):
        wid = lax.axis_index("s") * 2 + lax.axis_index("c")
        wbase = wid * n_per_w

        def chunk_body(ci, carry):
            base = wbase + ci * _CH
            pltpu.sync_copy(x0_hbm.at[pl.ds(base, _CH)], x0_v)
            pltpu.sync_copy(x1_hbm.at[pl.ds(base, _CH)], x1_v)

            for g in range(_G):
                for o in range(128 // _L):
                    s = g * 128 + o * _L
                    x0 = x0_v[pl.ds(s, _L)]
                    x1 = x1_v[pl.ds(s, _L)]
                    l0 = x0.astype(jnp.int32)
                    l1 = x1.astype(jnp.int32)
                    w = (l0.astype(jnp.float32) + 1.0 - x0) * (
                        l1.astype(jnp.float32) + 1.0 - x1)
                    i0 = jnp.minimum(l0, 63)
                    i1 = jnp.minimum(l1, 63)
                    idx_v[g, pl.ds(o * _L, _L)] = i0 * 64 + i1
                    w_v[pl.ds(s, _L)] = w

            copies = [
                pltpu.async_copy(
                    table_hbm.at[idx_v.at[g]],
                    rows_v.at[pl.ds(g * 128, 128)], sem)
                for g in range(_G)
            ]
            for cp in copies:
                cp.wait()

            def scale_body(jb, carry2):
                w16 = w_v[pl.ds(jb * _L, _L)]
                for r in range(_L):
                    j = jb * _L + r
                    wj = jnp.full((_L,), w16[r], dtype=jnp.float32)
                    for k in range(c // _L):
                        sl = pl.ds(k * _L, _L)
                        rows_v[j, sl] = rows_v[j, sl] * wj
                return carry2

            lax.fori_loop(0, _CH // _L, scale_body, 0)
            pltpu.sync_copy(rows_v, out_hbm.at[pl.ds(base, _CH)])
            return carry

        lax.fori_loop(0, n_chunks, chunk_body, 0)

    return body


def kernel(image, x):
    h, w, c = image.shape
    n = x.shape[0]
    table = image.reshape(h * w, c)
    x0 = x[:, 0]
    x1 = x[:, 1]
    info = plsc.get_sparse_core_info()
    nw = info.num_cores * info.num_subcores
    assert n % (nw * _CH) == 0
    return _interp_kernel(n, c, nw)(table, x0, x1)
